# baseline (speedup 1.0000x reference)
import functools

import jax
import jax.numpy as jnp
from jax import lax
from jax.experimental import pallas as pl
from jax.experimental.pallas import tpu as pltpu
from jax.experimental.pallas import tpu_sc as plsc

B, S, D = 1, 2048, 1024
FF = 3584
E = 8
TOP_K = 2

T = 128
MAXG = (TOP_K * S) // T + E
PAD_ROWS = MAXG * T

NW = 32
TPW = S // NW
FH = FF // 2
CH = 32
LANES = 16
WREP = 128


def _router_meta_body(h_ref, wg_ref, dst_ref, wrep_ref, texp_ref, tact_ref):
    h = h_ref[...]
    logits = jnp.dot(h, wg_ref[...], preferred_element_type=jnp.float32)
    m = jnp.max(logits, axis=1, keepdims=True)
    ex = jnp.exp(logits - m)
    p = ex / jnp.sum(ex, axis=1, keepdims=True)
    idx = lax.broadcasted_iota(jnp.int32, (S, E), 1)
    v0 = jnp.max(p, axis=1, keepdims=True)
    e0 = jnp.min(jnp.where(p == v0, idx, E), axis=1, keepdims=True)
    p1 = jnp.where(idx == e0, -jnp.inf, p)
    v1 = jnp.max(p1, axis=1, keepdims=True)
    e1 = jnp.min(jnp.where(p1 == v1, idx, E), axis=1, keepdims=True)
    s = v0 + v1
    w_a = jnp.concatenate([v0 / s, v1 / s], axis=0)
    wrep_ref[...] = jnp.broadcast_to(w_a, (TOP_K * S, WREP))

    e_a = jnp.concatenate([e0, e1], axis=0)
    idx2 = lax.broadcasted_iota(jnp.int32, (TOP_K * S, E), 1)
    oh = (e_a == idx2).astype(jnp.float32)

    RB = 512
    ri = lax.broadcasted_iota(jnp.int32, (RB, RB), 0)
    ci = lax.broadcasted_iota(jnp.int32, (RB, RB), 1)
    ltri = (ci < ri).astype(jnp.float32)
    carry = jnp.zeros((1, E), jnp.float32)
    ranks = []
    for b in range(TOP_K * S // RB):
        ohb = oh[b * RB:(b + 1) * RB, :]
        cb = jnp.dot(ltri, ohb, preferred_element_type=jnp.float32) + carry
        ranks.append(jnp.sum(cb * ohb, axis=1, keepdims=True))
        carry = carry + jnp.sum(ohb, axis=0, keepdims=True)
    rank_a = jnp.concatenate(ranks, axis=0)

    counts = carry
    ntiles = jnp.floor((counts + (T - 1)) / T)
    ri8 = lax.broadcasted_iota(jnp.int32, (E, E), 0)
    ci8 = lax.broadcasted_iota(jnp.int32, (E, E), 1)
    utri = (ri8 < ci8).astype(jnp.float32)
    cum_excl = jnp.dot(ntiles, utri, preferred_element_type=jnp.float32)
    poff = cum_excl * T
    poff_a = jnp.sum(oh * poff, axis=1, keepdims=True)
    dst_ref[...] = (rank_a + poff_a).astype(jnp.int32)

    ones_col = jnp.ones((TOP_K * S, 1), jnp.float32)
    counts_col = lax.dot_general(
        oh, ones_col, (((0,), (0,)), ((), ())),
        preferred_element_type=jnp.float32)
    ntiles_col = jnp.floor((counts_col + (T - 1)) / T)
    ltri8_inc = (ri8 >= ci8).astype(jnp.float32)
    cum_inc_col = jnp.dot(ltri8_inc, ntiles_col,
                          preferred_element_type=jnp.float32)
    cum_i = cum_inc_col.astype(jnp.int32)
    gi = lax.broadcasted_iota(jnp.int32, (E, MAXG), 1)
    texp_raw = jnp.sum((gi >= cum_i).astype(jnp.int32),
                       axis=0, keepdims=True)
    e_col = lax.broadcasted_iota(jnp.int32, (E, 1), 0)
    emax = jnp.max(jnp.where(ntiles_col > 0.5, e_col, -1))
    texp_ref[...] = jnp.minimum(texp_raw, emax)
    total = jnp.max(cum_i)
    tact_ref[...] = (gi[0:1, :] < total).astype(jnp.int32)


def _gmm_body(texp_ref, tact_ref, xs_ref, ws_ref, wup_ref, wgate_ref, wdown_ref,
              out_ref):
    g = pl.program_id(0)
    f = pl.program_id(1)
    active = tact_ref[0, g] == 1

    @pl.when(jnp.logical_not(active))
    def _():
        @pl.when(f == 0)
        def _():
            out_ref[...] = jnp.zeros_like(out_ref)

    @pl.when(active)
    def _():
        x = xs_ref[...].astype(jnp.bfloat16)
        up = jnp.dot(x, wup_ref[0].astype(jnp.bfloat16),
                     preferred_element_type=jnp.float32)
        gate = jnp.dot(x, wgate_ref[0].astype(jnp.bfloat16),
                       preferred_element_type=jnp.float32)
        z = (up * jax.nn.sigmoid(up) * gate).astype(jnp.bfloat16)
        part = jnp.dot(z, wdown_ref[0].astype(jnp.bfloat16),
                       preferred_element_type=jnp.float32)

        @pl.when(f == 0)
        def _():
            out_ref[...] = part

        @pl.when(f == 1)
        def _():
            out_ref[...] = (out_ref[...] + part) * ws_ref[...][:, 0:1]


@functools.cache
def _make_scatter_sc():
    mesh = plsc.VectorSubcoreMesh(core_axis_name="c", subcore_axis_name="s")

    @functools.partial(
        pl.kernel,
        out_type=[
            jax.ShapeDtypeStruct((PAD_ROWS, D), jnp.float32),
            jax.ShapeDtypeStruct((PAD_ROWS, WREP), jnp.float32),
        ],
        mesh=mesh,
        scratch_types=[
            pltpu.VMEM((TPW,), jnp.int32),
            pltpu.VMEM((TPW,), jnp.int32),
            pltpu.VMEM((TPW, D), jnp.float32),
            pltpu.VMEM((TPW, WREP), jnp.float32),
            pltpu.VMEM((TPW, WREP), jnp.float32),
            pltpu.SemaphoreType.DMA,
        ],
    )
    def _scatter_sc(h_hbm, dst_hbm, wrep_hbm, xs_hbm, ws_hbm,
                    idx0_v, idx1_v, rows_v, w0_v, w1_v, sem):
        wid = lax.axis_index("s") * 2 + lax.axis_index("c")
        base = wid * TPW
        pltpu.sync_copy(dst_hbm.at[pl.ds(base, TPW)], idx0_v)
        pltpu.sync_copy(dst_hbm.at[pl.ds(S + base, TPW)], idx1_v)
        pltpu.sync_copy(h_hbm.at[pl.ds(base, TPW)], rows_v)
        pltpu.sync_copy(wrep_hbm.at[pl.ds(base, TPW)], w0_v)
        pltpu.sync_copy(wrep_hbm.at[pl.ds(S + base, TPW)], w1_v)
        pltpu.async_copy(rows_v, xs_hbm.at[idx0_v], sem).wait()
        pltpu.async_copy(rows_v, xs_hbm.at[idx1_v], sem).wait()
        pltpu.async_copy(w0_v, ws_hbm.at[idx0_v], sem).wait()
        pltpu.async_copy(w1_v, ws_hbm.at[idx1_v], sem).wait()

    return _scatter_sc


@functools.cache
def _make_combine_sc():
    mesh = plsc.VectorSubcoreMesh(core_axis_name="c", subcore_axis_name="s")

    @functools.partial(
        pl.kernel,
        out_type=jax.ShapeDtypeStruct((S, D), jnp.float32),
        mesh=mesh,
        scratch_types=[
            pltpu.VMEM((CH,), jnp.int32),
            pltpu.VMEM((CH,), jnp.int32),
            pltpu.VMEM((CH, D), jnp.float32),
            pltpu.VMEM((CH, D), jnp.float32),
            pltpu.SemaphoreType.DMA,
        ],
    )
    def _combine_sc(y_hbm, dst_hbm, out_hbm, i0, i1, abuf, bbuf, sem):
        wid = lax.axis_index("s") * 2 + lax.axis_index("c")
        for ch in range(TPW // CH):
            base = wid * TPW + ch * CH
            pltpu.sync_copy(dst_hbm.at[pl.ds(base, CH)], i0)
            pltpu.sync_copy(dst_hbm.at[pl.ds(S + base, CH)], i1)
            pltpu.async_copy(y_hbm.at[i0], abuf, sem).wait()
            pltpu.async_copy(y_hbm.at[i1], bbuf, sem).wait()

            def row_body(r, _):
                def col_body(c, _):
                    av = abuf[r, pl.ds(c * LANES, LANES)]
                    bv = bbuf[r, pl.ds(c * LANES, LANES)]
                    abuf[r, pl.ds(c * LANES, LANES)] = av + bv
                    return 0

                lax.fori_loop(0, D // LANES, col_body, 0)
                return 0

            lax.fori_loop(0, CH, row_body, 0)
            pltpu.sync_copy(abuf, out_hbm.at[pl.ds(base, CH)])

    return _combine_sc


@jax.jit
def _run(h2d, Wg, W_up, W_gate, W_down):
    dst, wrep, texp, tact = pl.pallas_call(
        _router_meta_body,
        out_shape=[
            jax.ShapeDtypeStruct((TOP_K * S, 1), jnp.int32),
            jax.ShapeDtypeStruct((TOP_K * S, WREP), jnp.float32),
            jax.ShapeDtypeStruct((1, MAXG), jnp.int32),
            jax.ShapeDtypeStruct((1, MAXG), jnp.int32),
        ],
    )(h2d, Wg)
    dst_flat = dst.reshape(TOP_K * S)

    x_sorted, ws_sorted = _make_scatter_sc()(h2d, dst_flat, wrep)

    y = pl.pallas_call(
        _gmm_body,
        grid_spec=pltpu.PrefetchScalarGridSpec(
            num_scalar_prefetch=2,
            grid=(MAXG, 2),
            in_specs=[
                pl.BlockSpec((T, D), lambda g, f, texp_r, tact_r: (g, 0)),
                pl.BlockSpec((T, WREP), lambda g, f, texp_r, tact_r: (g, 0)),
                pl.BlockSpec(
                    (1, D, FH),
                    lambda g, f, texp_r, tact_r: (texp_r[0, g], 0, f),
                ),
                pl.BlockSpec(
                    (1, D, FH),
                    lambda g, f, texp_r, tact_r: (texp_r[0, g], 0, f),
                ),
                pl.BlockSpec(
                    (1, FH, D),
                    lambda g, f, texp_r, tact_r: (texp_r[0, g], f, 0),
                ),
            ],
            out_specs=pl.BlockSpec((T, D), lambda g, f, texp_r, tact_r: (g, 0)),
        ),
        out_shape=jax.ShapeDtypeStruct((PAD_ROWS, D), jnp.float32),
    )(texp, tact, x_sorted, ws_sorted, W_up, W_gate, W_down)

    return _make_combine_sc()(y, dst_flat)


def kernel(hidden_states, Wg, W_up, W_gate, W_down):
    h2d = hidden_states.reshape(-1, D)
    out = _run(h2d, Wg, W_up, W_gate, W_down)
    return out.reshape(hidden_states.shape)


# speedup vs baseline: 1.7396x; 1.7396x over previous
import functools

import jax
import jax.numpy as jnp
from jax import lax
from jax.experimental import pallas as pl
from jax.experimental.pallas import tpu as pltpu
from jax.experimental.pallas import tpu_sc as plsc

B, S, D = 1, 2048, 1024
FF = 3584
E = 8
TOP_K = 2

T = 128
MAXG = (TOP_K * S) // T + E
PAD_ROWS = MAXG * T

NW = 32
TPW = S // NW
FH = FF // 2
CH = 16
LANES = 16
WREP = 128


def _router_meta_body(h_ref, wg_ref, dst_ref, wrep_ref, texp_ref, tact_ref):
    h = h_ref[...]
    logits = jnp.dot(h, wg_ref[...], preferred_element_type=jnp.float32)
    m = jnp.max(logits, axis=1, keepdims=True)
    ex = jnp.exp(logits - m)
    p = ex / jnp.sum(ex, axis=1, keepdims=True)
    idx = lax.broadcasted_iota(jnp.int32, (S, E), 1)
    v0 = jnp.max(p, axis=1, keepdims=True)
    e0 = jnp.min(jnp.where(p == v0, idx, E), axis=1, keepdims=True)
    p1 = jnp.where(idx == e0, -jnp.inf, p)
    v1 = jnp.max(p1, axis=1, keepdims=True)
    e1 = jnp.min(jnp.where(p1 == v1, idx, E), axis=1, keepdims=True)
    s = v0 + v1
    w_a = jnp.concatenate([v0 / s, v1 / s], axis=0)
    wrep_ref[...] = jnp.broadcast_to(w_a, (TOP_K * S, WREP))

    e_a = jnp.concatenate([e0, e1], axis=0)
    idx2 = lax.broadcasted_iota(jnp.int32, (TOP_K * S, E), 1)
    oh = (e_a == idx2).astype(jnp.float32)

    RB = 512
    ri = lax.broadcasted_iota(jnp.int32, (RB, RB), 0)
    ci = lax.broadcasted_iota(jnp.int32, (RB, RB), 1)
    ltri = (ci < ri).astype(jnp.float32)
    carry = jnp.zeros((1, E), jnp.float32)
    ranks = []
    for b in range(TOP_K * S // RB):
        ohb = oh[b * RB:(b + 1) * RB, :]
        cb = jnp.dot(ltri, ohb, preferred_element_type=jnp.float32) + carry
        ranks.append(jnp.sum(cb * ohb, axis=1, keepdims=True))
        carry = carry + jnp.sum(ohb, axis=0, keepdims=True)
    rank_a = jnp.concatenate(ranks, axis=0)

    counts = carry
    ntiles = jnp.floor((counts + (T - 1)) / T)
    ri8 = lax.broadcasted_iota(jnp.int32, (E, E), 0)
    ci8 = lax.broadcasted_iota(jnp.int32, (E, E), 1)
    utri = (ri8 < ci8).astype(jnp.float32)
    cum_excl = jnp.dot(ntiles, utri, preferred_element_type=jnp.float32)
    poff = cum_excl * T
    poff_a = jnp.sum(oh * poff, axis=1, keepdims=True)
    dst = (rank_a + poff_a).astype(jnp.int32)
    dst_ref[...] = jnp.concatenate([dst, dst + PAD_ROWS], axis=0)

    ones_col = jnp.ones((TOP_K * S, 1), jnp.float32)
    counts_col = lax.dot_general(
        oh, ones_col, (((0,), (0,)), ((), ())),
        preferred_element_type=jnp.float32)
    ntiles_col = jnp.floor((counts_col + (T - 1)) / T)
    ltri8_inc = (ri8 >= ci8).astype(jnp.float32)
    cum_inc_col = jnp.dot(ltri8_inc, ntiles_col,
                          preferred_element_type=jnp.float32)
    cum_i = cum_inc_col.astype(jnp.int32)
    gi = lax.broadcasted_iota(jnp.int32, (E, MAXG), 1)
    texp_raw = jnp.sum((gi >= cum_i).astype(jnp.int32),
                       axis=0, keepdims=True)
    e_col = lax.broadcasted_iota(jnp.int32, (E, 1), 0)
    emax = jnp.max(jnp.where(ntiles_col > 0.5, e_col, -1))
    texp_ref[...] = jnp.minimum(texp_raw, emax)
    total = jnp.max(cum_i)
    tact_ref[...] = (gi[0:1, :] < total).astype(jnp.int32)


def _gmm_body(texp_ref, tact_ref, xs_ref, ws_ref, wup_ref, wgate_ref, wdown_ref,
              out_ref):
    active = tact_ref[0, pl.program_id(1)] == 1

    @pl.when(jnp.logical_not(active))
    def _():
        out_ref[...] = jnp.zeros_like(out_ref)

    @pl.when(active)
    def _():
        x = xs_ref[...].astype(jnp.bfloat16)
        up = jnp.dot(x, wup_ref[0].astype(jnp.bfloat16),
                     preferred_element_type=jnp.float32)
        gate = jnp.dot(x, wgate_ref[0].astype(jnp.bfloat16),
                       preferred_element_type=jnp.float32)
        z = (up * jax.nn.sigmoid(up) * gate).astype(jnp.bfloat16)
        out_ref[...] = jnp.dot(
            z, wdown_ref[0].astype(jnp.bfloat16),
            preferred_element_type=jnp.float32
        ) * ws_ref[...][:, 0:1]


@functools.cache
def _make_scatter_sc():
    mesh = plsc.VectorSubcoreMesh(core_axis_name="c", subcore_axis_name="s")

    @functools.partial(
        pl.kernel,
        out_type=[
            jax.ShapeDtypeStruct((PAD_ROWS, D), jnp.float32),
            jax.ShapeDtypeStruct((PAD_ROWS, WREP), jnp.float32),
        ],
        mesh=mesh,
        scratch_types=[
            pltpu.VMEM((TPW,), jnp.int32),
            pltpu.VMEM((TPW,), jnp.int32),
            pltpu.VMEM((TPW, D), jnp.float32),
            pltpu.VMEM((TPW, WREP), jnp.float32),
            pltpu.VMEM((TPW, WREP), jnp.float32),
            pltpu.SemaphoreType.DMA,
        ],
    )
    def _scatter_sc(h_hbm, dst_hbm, wrep_hbm, xs_hbm, ws_hbm,
                    idx0_v, idx1_v, rows_v, w0_v, w1_v, sem):
        wid = lax.axis_index("s") * 2 + lax.axis_index("c")
        base = wid * TPW
        pltpu.sync_copy(dst_hbm.at[pl.ds(base, TPW)], idx0_v)
        pltpu.sync_copy(dst_hbm.at[pl.ds(S + base, TPW)], idx1_v)
        pltpu.sync_copy(h_hbm.at[pl.ds(base, TPW)], rows_v)
        pltpu.sync_copy(wrep_hbm.at[pl.ds(base, TPW)], w0_v)
        pltpu.sync_copy(wrep_hbm.at[pl.ds(S + base, TPW)], w1_v)
        pltpu.async_copy(rows_v, xs_hbm.at[idx0_v], sem).wait()
        pltpu.async_copy(rows_v, xs_hbm.at[idx1_v], sem).wait()
        pltpu.async_copy(w0_v, ws_hbm.at[idx0_v], sem).wait()
        pltpu.async_copy(w1_v, ws_hbm.at[idx1_v], sem).wait()

    return _scatter_sc


@functools.cache
def _make_combine_sc():
    mesh = plsc.VectorSubcoreMesh(core_axis_name="c", subcore_axis_name="s")

    @functools.partial(
        pl.kernel,
        out_type=jax.ShapeDtypeStruct((S, D), jnp.float32),
        mesh=mesh,
        scratch_types=[
            pltpu.VMEM((CH,), jnp.int32),
            pltpu.VMEM((CH,), jnp.int32),
            pltpu.VMEM((CH,), jnp.int32),
            pltpu.VMEM((CH,), jnp.int32),
            pltpu.VMEM((CH, D), jnp.float32),
            pltpu.VMEM((CH, D), jnp.float32),
            pltpu.VMEM((CH, D), jnp.float32),
            pltpu.VMEM((CH, D), jnp.float32),
            pltpu.SemaphoreType.DMA,
        ],
    )
    def _combine_sc(y_hbm, dst_hbm, out_hbm, i0, i1, i2, i3,
                    abuf, bbuf, cbuf, dbuf, sem):
        wid = lax.axis_index("s") * 2 + lax.axis_index("c")
        for ch in range(TPW // CH):
            base = wid * TPW + ch * CH
            pltpu.sync_copy(dst_hbm.at[pl.ds(base, CH)], i0)
            pltpu.sync_copy(dst_hbm.at[pl.ds(S + base, CH)], i1)
            pltpu.sync_copy(dst_hbm.at[pl.ds(2 * S + base, CH)], i2)
            pltpu.sync_copy(dst_hbm.at[pl.ds(3 * S + base, CH)], i3)
            pltpu.async_copy(y_hbm.at[i0], abuf, sem).wait()
            pltpu.async_copy(y_hbm.at[i1], bbuf, sem).wait()
            pltpu.async_copy(y_hbm.at[i2], cbuf, sem).wait()
            pltpu.async_copy(y_hbm.at[i3], dbuf, sem).wait()

            def row_body(r, _):
                def col_body(c, _):
                    av = abuf[r, pl.ds(c * LANES, LANES)]
                    bv = bbuf[r, pl.ds(c * LANES, LANES)]
                    cv = cbuf[r, pl.ds(c * LANES, LANES)]
                    dv = dbuf[r, pl.ds(c * LANES, LANES)]
                    abuf[r, pl.ds(c * LANES, LANES)] = (av + bv) + (cv + dv)
                    return 0

                lax.fori_loop(0, D // LANES, col_body, 0)
                return 0

            lax.fori_loop(0, CH, row_body, 0)
            pltpu.sync_copy(abuf, out_hbm.at[pl.ds(base, CH)])

    return _combine_sc


@jax.jit
def _run(h2d, Wg, W_up, W_gate, W_down):
    dst, wrep, texp, tact = pl.pallas_call(
        _router_meta_body,
        out_shape=[
            jax.ShapeDtypeStruct((2 * TOP_K * S, 1), jnp.int32),
            jax.ShapeDtypeStruct((TOP_K * S, WREP), jnp.float32),
            jax.ShapeDtypeStruct((1, MAXG), jnp.int32),
            jax.ShapeDtypeStruct((1, MAXG), jnp.int32),
        ],
    )(h2d, Wg)
    dst_flat = dst.reshape(2 * TOP_K * S)

    x_sorted, ws_sorted = _make_scatter_sc()(h2d, dst_flat, wrep)

    y = pl.pallas_call(
        _gmm_body,
        grid_spec=pltpu.PrefetchScalarGridSpec(
            num_scalar_prefetch=2,
            grid=(2, MAXG),
            in_specs=[
                pl.BlockSpec((T, D), lambda f, g, texp_r, tact_r: (g, 0)),
                pl.BlockSpec((T, WREP), lambda f, g, texp_r, tact_r: (g, 0)),
                pl.BlockSpec(
                    (1, D, FH),
                    lambda f, g, texp_r, tact_r: (texp_r[0, g], 0, f),
                ),
                pl.BlockSpec(
                    (1, D, FH),
                    lambda f, g, texp_r, tact_r: (texp_r[0, g], 0, f),
                ),
                pl.BlockSpec(
                    (1, FH, D),
                    lambda f, g, texp_r, tact_r: (texp_r[0, g], f, 0),
                ),
            ],
            out_specs=pl.BlockSpec(
                (T, D), lambda f, g, texp_r, tact_r: (f * MAXG + g, 0)
            ),
        ),
        out_shape=jax.ShapeDtypeStruct((2 * PAD_ROWS, D), jnp.float32),
    )(texp, tact, x_sorted, ws_sorted, W_up, W_gate, W_down)

    return _make_combine_sc()(y, dst_flat)


def kernel(hidden_states, Wg, W_up, W_gate, W_down):
    h2d = hidden_states.reshape(-1, D)
    out = _run(h2d, Wg, W_up, W_gate, W_down)
    return out.reshape(hidden_states.shape)


# speedup vs baseline: 1.8680x; 1.0738x over previous
import functools

import jax
import jax.numpy as jnp
from jax import lax
from jax.experimental import pallas as pl
from jax.experimental.pallas import tpu as pltpu
from jax.experimental.pallas import tpu_sc as plsc

B, S, D = 1, 2048, 1024
FF = 3584
E = 8
TOP_K = 2

T = 128
MAXG = (TOP_K * S) // T + E
PAD_ROWS = MAXG * T

NW = 32
TPW = S // NW
FH = FF // 2
CH = 16
LANES = 16
WREP = 128


def _router_meta_body(h_ref, wg_ref, dst_ref, wrep_ref, texp_ref, tact_ref,
                      tslot_ref, tnext_ref):
    h = h_ref[...]
    logits = jnp.dot(h, wg_ref[...], preferred_element_type=jnp.float32)
    m = jnp.max(logits, axis=1, keepdims=True)
    ex = jnp.exp(logits - m)
    p = ex / jnp.sum(ex, axis=1, keepdims=True)
    idx = lax.broadcasted_iota(jnp.int32, (S, E), 1)
    v0 = jnp.max(p, axis=1, keepdims=True)
    e0 = jnp.min(jnp.where(p == v0, idx, E), axis=1, keepdims=True)
    p1 = jnp.where(idx == e0, -jnp.inf, p)
    v1 = jnp.max(p1, axis=1, keepdims=True)
    e1 = jnp.min(jnp.where(p1 == v1, idx, E), axis=1, keepdims=True)
    s = v0 + v1
    w_a = jnp.concatenate([v0 / s, v1 / s], axis=0)
    wrep_ref[...] = jnp.broadcast_to(w_a, (TOP_K * S, WREP))

    e_a = jnp.concatenate([e0, e1], axis=0)
    idx2 = lax.broadcasted_iota(jnp.int32, (TOP_K * S, E), 1)
    oh = (e_a == idx2).astype(jnp.float32)

    RB = 512
    ri = lax.broadcasted_iota(jnp.int32, (RB, RB), 0)
    ci = lax.broadcasted_iota(jnp.int32, (RB, RB), 1)
    ltri = (ci < ri).astype(jnp.float32)
    carry = jnp.zeros((1, E), jnp.float32)
    ranks = []
    for b in range(TOP_K * S // RB):
        ohb = oh[b * RB:(b + 1) * RB, :]
        cb = jnp.dot(ltri, ohb, preferred_element_type=jnp.float32) + carry
        ranks.append(jnp.sum(cb * ohb, axis=1, keepdims=True))
        carry = carry + jnp.sum(ohb, axis=0, keepdims=True)
    rank_a = jnp.concatenate(ranks, axis=0)

    counts = carry
    ntiles = jnp.floor((counts + (T - 1)) / T)
    ri8 = lax.broadcasted_iota(jnp.int32, (E, E), 0)
    ci8 = lax.broadcasted_iota(jnp.int32, (E, E), 1)
    utri = (ri8 < ci8).astype(jnp.float32)
    cum_excl = jnp.dot(ntiles, utri, preferred_element_type=jnp.float32)
    poff = cum_excl * T
    poff_a = jnp.sum(oh * poff, axis=1, keepdims=True)
    dst = (rank_a + poff_a).astype(jnp.int32)
    dst_ref[...] = jnp.concatenate([dst, dst + PAD_ROWS], axis=0)

    ones_col = jnp.ones((TOP_K * S, 1), jnp.float32)
    counts_col = lax.dot_general(
        oh, ones_col, (((0,), (0,)), ((), ())),
        preferred_element_type=jnp.float32)
    ntiles_col = jnp.floor((counts_col + (T - 1)) / T)
    ltri8_inc = (ri8 >= ci8).astype(jnp.float32)
    cum_inc_col = jnp.dot(ltri8_inc, ntiles_col,
                          preferred_element_type=jnp.float32)
    cum_i = cum_inc_col.astype(jnp.int32)
    gi = lax.broadcasted_iota(jnp.int32, (E, MAXG), 1)
    texp_raw = jnp.sum((gi >= cum_i).astype(jnp.int32),
                       axis=0, keepdims=True)
    e_col = lax.broadcasted_iota(jnp.int32, (E, 1), 0)
    emax = jnp.max(jnp.where(ntiles_col > 0.5, e_col, -1))
    texp_row = jnp.minimum(texp_raw, emax)
    texp_ref[...] = texp_row
    total = jnp.max(cum_i)
    tact_ref[...] = (gi[0:1, :] < total).astype(jnp.int32)

    prev_row = jnp.concatenate([texp_row[:, :1], texp_row[:, :-1]], axis=1)
    c_run = jnp.logical_or(gi[0:1, :] == 0,
                           texp_row != prev_row).astype(jnp.float32)
    mi = lax.broadcasted_iota(jnp.int32, (MAXG, MAXG), 0)
    mj = lax.broadcasted_iota(jnp.int32, (MAXG, MAXG), 1)
    ltri_m = (mi <= mj).astype(jnp.float32)
    rid = jnp.dot(c_run, ltri_m, preferred_element_type=jnp.float32) - 1.0
    tslot_ref[...] = (rid - 2.0 * jnp.floor(rid * 0.5)).astype(jnp.int32)
    ecnt = jnp.broadcast_to(counts, (E, E))
    cand = jnp.where(jnp.logical_and(ci8 > ri8, ecnt > 0.5), ci8, 2 * E)
    nxt = jnp.min(cand, axis=1, keepdims=True)
    nxt = jnp.where(nxt == 2 * E, -1, nxt)
    e_col8b = lax.broadcasted_iota(jnp.int32, (E, 1), 0)
    onehot_t = jnp.broadcast_to(texp_row, (E, MAXG)) == e_col8b
    tnext_ref[...] = jnp.sum(
        jnp.where(onehot_t, jnp.broadcast_to(nxt, (E, MAXG)), 0),
        axis=0, keepdims=True).astype(jnp.int32)


def _gmm_body(texp_ref, tslot_ref, tnext_ref, tact_ref,
              xs_ref, ws_ref, wup_hbm, wgate_hbm, wdown_hbm, out_ref,
              wup_b, wgate_b, wdown_b, sems):
    f = pl.program_id(0)
    g = pl.program_id(1)
    e = texp_ref[0, g]
    slot = tslot_ref[0, g]
    en = tnext_ref[0, g]
    active = tact_ref[0, g] == 1
    prev = texp_ref[0, jnp.maximum(g - 1, 0)]
    first = jnp.logical_or(g == 0, e != prev)

    def copies(eidx, sidx):
        fo = f * FH
        return [
            pltpu.make_async_copy(wup_hbm.at[eidx, :, pl.ds(fo, FH)],
                                  wup_b.at[sidx], sems.at[0, sidx]),
            pltpu.make_async_copy(wgate_hbm.at[eidx, :, pl.ds(fo, FH)],
                                  wgate_b.at[sidx], sems.at[1, sidx]),
            pltpu.make_async_copy(wdown_hbm.at[eidx, pl.ds(fo, FH), :],
                                  wdown_b.at[sidx], sems.at[2, sidx]),
        ]

    @pl.when(g == 0)
    def _():
        for cp in copies(e, slot):
            cp.start()

    @pl.when(jnp.logical_and(first, en >= 0))
    def _():
        for cp in copies(en, 1 - slot):
            cp.start()

    @pl.when(first)
    def _():
        for cp in copies(e, slot):
            cp.wait()

    @pl.when(jnp.logical_not(active))
    def _():
        out_ref[...] = jnp.zeros_like(out_ref)

    @pl.when(active)
    def _():
        x = xs_ref[...].astype(jnp.bfloat16)
        up = jnp.dot(x, wup_b[slot].astype(jnp.bfloat16),
                     preferred_element_type=jnp.float32)
        gate = jnp.dot(x, wgate_b[slot].astype(jnp.bfloat16),
                       preferred_element_type=jnp.float32)
        z = (up * jax.nn.sigmoid(up) * gate).astype(jnp.bfloat16)
        out_ref[...] = jnp.dot(
            z, wdown_b[slot].astype(jnp.bfloat16),
            preferred_element_type=jnp.float32
        ) * ws_ref[...][:, 0:1]


@functools.cache
def _make_scatter_sc():
    mesh = plsc.VectorSubcoreMesh(core_axis_name="c", subcore_axis_name="s")

    @functools.partial(
        pl.kernel,
        out_type=[
            jax.ShapeDtypeStruct((PAD_ROWS, D), jnp.float32),
            jax.ShapeDtypeStruct((PAD_ROWS, WREP), jnp.float32),
        ],
        mesh=mesh,
        scratch_types=[
            pltpu.VMEM((TPW,), jnp.int32),
            pltpu.VMEM((TPW,), jnp.int32),
            pltpu.VMEM((TPW, D), jnp.float32),
            pltpu.VMEM((TPW, WREP), jnp.float32),
            pltpu.VMEM((TPW, WREP), jnp.float32),
            pltpu.SemaphoreType.DMA,
        ],
    )
    def _scatter_sc(h_hbm, dst_hbm, wrep_hbm, xs_hbm, ws_hbm,
                    idx0_v, idx1_v, rows_v, w0_v, w1_v, sem):
        wid = lax.axis_index("s") * 2 + lax.axis_index("c")
        base = wid * TPW
        pltpu.sync_copy(dst_hbm.at[pl.ds(base, TPW)], idx0_v)
        pltpu.sync_copy(dst_hbm.at[pl.ds(S + base, TPW)], idx1_v)
        pltpu.sync_copy(h_hbm.at[pl.ds(base, TPW)], rows_v)
        pltpu.sync_copy(wrep_hbm.at[pl.ds(base, TPW)], w0_v)
        pltpu.sync_copy(wrep_hbm.at[pl.ds(S + base, TPW)], w1_v)
        pltpu.async_copy(rows_v, xs_hbm.at[idx0_v], sem).wait()
        pltpu.async_copy(rows_v, xs_hbm.at[idx1_v], sem).wait()
        pltpu.async_copy(w0_v, ws_hbm.at[idx0_v], sem).wait()
        pltpu.async_copy(w1_v, ws_hbm.at[idx1_v], sem).wait()

    return _scatter_sc


@functools.cache
def _make_combine_sc():
    mesh = plsc.VectorSubcoreMesh(core_axis_name="c", subcore_axis_name="s")

    @functools.partial(
        pl.kernel,
        out_type=jax.ShapeDtypeStruct((S, D), jnp.float32),
        mesh=mesh,
        scratch_types=[
            pltpu.VMEM((CH,), jnp.int32),
            pltpu.VMEM((CH,), jnp.int32),
            pltpu.VMEM((CH,), jnp.int32),
            pltpu.VMEM((CH,), jnp.int32),
            pltpu.VMEM((CH, D), jnp.float32),
            pltpu.VMEM((CH, D), jnp.float32),
            pltpu.VMEM((CH, D), jnp.float32),
            pltpu.VMEM((CH, D), jnp.float32),
            pltpu.SemaphoreType.DMA,
        ],
    )
    def _combine_sc(y_hbm, dst_hbm, out_hbm, i0, i1, i2, i3,
                    abuf, bbuf, cbuf, dbuf, sem):
        wid = lax.axis_index("s") * 2 + lax.axis_index("c")
        for ch in range(TPW // CH):
            base = wid * TPW + ch * CH
            pltpu.sync_copy(dst_hbm.at[pl.ds(base, CH)], i0)
            pltpu.sync_copy(dst_hbm.at[pl.ds(S + base, CH)], i1)
            pltpu.sync_copy(dst_hbm.at[pl.ds(2 * S + base, CH)], i2)
            pltpu.sync_copy(dst_hbm.at[pl.ds(3 * S + base, CH)], i3)
            pltpu.async_copy(y_hbm.at[i0], abuf, sem).wait()
            pltpu.async_copy(y_hbm.at[i1], bbuf, sem).wait()
            pltpu.async_copy(y_hbm.at[i2], cbuf, sem).wait()
            pltpu.async_copy(y_hbm.at[i3], dbuf, sem).wait()

            def row_body(r, _):
                def col_body(c, _):
                    av = abuf[r, pl.ds(c * LANES, LANES)]
                    bv = bbuf[r, pl.ds(c * LANES, LANES)]
                    cv = cbuf[r, pl.ds(c * LANES, LANES)]
                    dv = dbuf[r, pl.ds(c * LANES, LANES)]
                    abuf[r, pl.ds(c * LANES, LANES)] = (av + bv) + (cv + dv)
                    return 0

                lax.fori_loop(0, D // LANES, col_body, 0)
                return 0

            lax.fori_loop(0, CH, row_body, 0)
            pltpu.sync_copy(abuf, out_hbm.at[pl.ds(base, CH)])

    return _combine_sc


@jax.jit
def _run(h2d, Wg, W_up, W_gate, W_down):
    dst, wrep, texp, tact, tslot, tnext = pl.pallas_call(
        _router_meta_body,
        out_shape=[
            jax.ShapeDtypeStruct((2 * TOP_K * S, 1), jnp.int32),
            jax.ShapeDtypeStruct((TOP_K * S, WREP), jnp.float32),
            jax.ShapeDtypeStruct((1, MAXG), jnp.int32),
            jax.ShapeDtypeStruct((1, MAXG), jnp.int32),
            jax.ShapeDtypeStruct((1, MAXG), jnp.int32),
            jax.ShapeDtypeStruct((1, MAXG), jnp.int32),
        ],
    )(h2d, Wg)
    dst_flat = dst.reshape(2 * TOP_K * S)

    x_sorted, ws_sorted = _make_scatter_sc()(h2d, dst_flat, wrep)

    y = pl.pallas_call(
        _gmm_body,
        grid_spec=pltpu.PrefetchScalarGridSpec(
            num_scalar_prefetch=4,
            grid=(2, MAXG),
            in_specs=[
                pl.BlockSpec((T, D), lambda f, g, *_: (g, 0)),
                pl.BlockSpec((T, WREP), lambda f, g, *_: (g, 0)),
                pl.BlockSpec(memory_space=pltpu.MemorySpace.HBM),
                pl.BlockSpec(memory_space=pltpu.MemorySpace.HBM),
                pl.BlockSpec(memory_space=pltpu.MemorySpace.HBM),
            ],
            out_specs=pl.BlockSpec(
                (T, D), lambda f, g, *_: (f * MAXG + g, 0)
            ),
            scratch_shapes=[
                pltpu.VMEM((2, D, FH), jnp.float32),
                pltpu.VMEM((2, D, FH), jnp.float32),
                pltpu.VMEM((2, FH, D), jnp.float32),
                pltpu.SemaphoreType.DMA((3, 2)),
            ],
        ),
        out_shape=jax.ShapeDtypeStruct((2 * PAD_ROWS, D), jnp.float32),
    )(texp, tslot, tnext, tact, x_sorted, ws_sorted, W_up, W_gate, W_down)

    return _make_combine_sc()(y, dst_flat)


def kernel(hidden_states, Wg, W_up, W_gate, W_down):
    h2d = hidden_states.reshape(-1, D)
    out = _run(h2d, Wg, W_up, W_gate, W_down)
    return out.reshape(hidden_states.shape)


# speedup vs baseline: 1.9148x; 1.0250x over previous
import functools

import jax
import jax.numpy as jnp
from jax import lax
from jax.experimental import pallas as pl
from jax.experimental.pallas import tpu as pltpu
from jax.experimental.pallas import tpu_sc as plsc

B, S, D = 1, 2048, 1024
FF = 3584
E = 8
TOP_K = 2

T = 128
MAXG = (TOP_K * S) // T + E
PAD_ROWS = MAXG * T

NW = 32
TPW = S // NW
FH = FF // 2
CH = 16
LANES = 16
WREP = 128


def _router_meta_body(h_ref, wg_ref, dst_ref, wrep_ref, texp_ref, tact_ref,
                      tslot_ref, tnext_ref):
    h = h_ref[...]
    logits = jnp.dot(h, wg_ref[...], preferred_element_type=jnp.float32)
    m = jnp.max(logits, axis=1, keepdims=True)
    ex = jnp.exp(logits - m)
    p = ex / jnp.sum(ex, axis=1, keepdims=True)
    idx = lax.broadcasted_iota(jnp.int32, (S, E), 1)
    v0 = jnp.max(p, axis=1, keepdims=True)
    e0 = jnp.min(jnp.where(p == v0, idx, E), axis=1, keepdims=True)
    p1 = jnp.where(idx == e0, -jnp.inf, p)
    v1 = jnp.max(p1, axis=1, keepdims=True)
    e1 = jnp.min(jnp.where(p1 == v1, idx, E), axis=1, keepdims=True)
    s = v0 + v1
    w_a = jnp.concatenate([v0 / s, v1 / s], axis=0)
    wrep_ref[...] = jnp.broadcast_to(w_a, (TOP_K * S, WREP))

    e_a = jnp.concatenate([e0, e1], axis=0)
    idx2 = lax.broadcasted_iota(jnp.int32, (TOP_K * S, E), 1)
    oh = (e_a == idx2).astype(jnp.float32)

    RB = 512
    ri = lax.broadcasted_iota(jnp.int32, (RB, RB), 0)
    ci = lax.broadcasted_iota(jnp.int32, (RB, RB), 1)
    ltri = (ci < ri).astype(jnp.float32)
    carry = jnp.zeros((1, E), jnp.float32)
    ranks = []
    for b in range(TOP_K * S // RB):
        ohb = oh[b * RB:(b + 1) * RB, :]
        cb = jnp.dot(ltri, ohb, preferred_element_type=jnp.float32) + carry
        ranks.append(jnp.sum(cb * ohb, axis=1, keepdims=True))
        carry = carry + jnp.sum(ohb, axis=0, keepdims=True)
    rank_a = jnp.concatenate(ranks, axis=0)

    counts = carry
    ntiles = jnp.floor((counts + (T - 1)) / T)
    ri8 = lax.broadcasted_iota(jnp.int32, (E, E), 0)
    ci8 = lax.broadcasted_iota(jnp.int32, (E, E), 1)
    utri = (ri8 < ci8).astype(jnp.float32)
    cum_excl = jnp.dot(ntiles, utri, preferred_element_type=jnp.float32)
    poff = cum_excl * T
    poff_a = jnp.sum(oh * poff, axis=1, keepdims=True)
    dst = (rank_a + poff_a).astype(jnp.int32)
    dst_ref[...] = jnp.concatenate([dst, dst + PAD_ROWS], axis=0)

    ones_col = jnp.ones((TOP_K * S, 1), jnp.float32)
    counts_col = lax.dot_general(
        oh, ones_col, (((0,), (0,)), ((), ())),
        preferred_element_type=jnp.float32)
    ntiles_col = jnp.floor((counts_col + (T - 1)) / T)
    ltri8_inc = (ri8 >= ci8).astype(jnp.float32)
    cum_inc_col = jnp.dot(ltri8_inc, ntiles_col,
                          preferred_element_type=jnp.float32)
    cum_i = cum_inc_col.astype(jnp.int32)
    gi = lax.broadcasted_iota(jnp.int32, (E, MAXG), 1)
    texp_raw = jnp.sum((gi >= cum_i).astype(jnp.int32),
                       axis=0, keepdims=True)
    e_col = lax.broadcasted_iota(jnp.int32, (E, 1), 0)
    emax = jnp.max(jnp.where(ntiles_col > 0.5, e_col, -1))
    texp_row = jnp.minimum(texp_raw, emax)
    texp_ref[...] = texp_row
    total = jnp.max(cum_i)
    tact_ref[...] = (gi[0:1, :] < total).astype(jnp.int32)

    prev_row = jnp.concatenate([texp_row[:, :1], texp_row[:, :-1]], axis=1)
    c_run = jnp.logical_or(gi[0:1, :] == 0,
                           texp_row != prev_row).astype(jnp.float32)
    mi = lax.broadcasted_iota(jnp.int32, (MAXG, MAXG), 0)
    mj = lax.broadcasted_iota(jnp.int32, (MAXG, MAXG), 1)
    ltri_m = (mi <= mj).astype(jnp.float32)
    rid = jnp.dot(c_run, ltri_m, preferred_element_type=jnp.float32) - 1.0
    tslot_ref[...] = (rid - 2.0 * jnp.floor(rid * 0.5)).astype(jnp.int32)
    ecnt = jnp.broadcast_to(counts, (E, E))
    cand = jnp.where(jnp.logical_and(ci8 > ri8, ecnt > 0.5), ci8, 2 * E)
    nxt = jnp.min(cand, axis=1, keepdims=True)
    nxt = jnp.where(nxt == 2 * E, -1, nxt)
    e_col8b = lax.broadcasted_iota(jnp.int32, (E, 1), 0)
    onehot_t = jnp.broadcast_to(texp_row, (E, MAXG)) == e_col8b
    tnext_ref[...] = jnp.sum(
        jnp.where(onehot_t, jnp.broadcast_to(nxt, (E, MAXG)), 0),
        axis=0, keepdims=True).astype(jnp.int32)


def _gmm_body(texp_ref, tslot_ref, tnext_ref, tact_ref,
              xs_ref, ws_ref, wup_hbm, wgate_hbm, wdown_hbm, out_ref,
              wup_b, wgate_b, wdown_b, sems):
    f = pl.program_id(0)
    g = pl.program_id(1)
    e = texp_ref[0, g]
    slot = tslot_ref[0, g]
    en = tnext_ref[0, g]
    active = tact_ref[0, g] == 1
    prev = texp_ref[0, jnp.maximum(g - 1, 0)]
    first = jnp.logical_or(g == 0, e != prev)

    def copies(eidx, sidx):
        fo = f * FH
        return [
            pltpu.make_async_copy(wup_hbm.at[eidx, :, pl.ds(fo, FH)],
                                  wup_b.at[sidx], sems.at[0, sidx]),
            pltpu.make_async_copy(wgate_hbm.at[eidx, :, pl.ds(fo, FH)],
                                  wgate_b.at[sidx], sems.at[1, sidx]),
            pltpu.make_async_copy(wdown_hbm.at[eidx, pl.ds(fo, FH), :],
                                  wdown_b.at[sidx], sems.at[2, sidx]),
        ]

    @pl.when(g == 0)
    def _():
        for cp in copies(e, slot):
            cp.start()

    @pl.when(jnp.logical_and(first, en >= 0))
    def _():
        for cp in copies(en, 1 - slot):
            cp.start()

    @pl.when(first)
    def _():
        for cp in copies(e, slot):
            cp.wait()

    @pl.when(jnp.logical_not(active))
    def _():
        out_ref[...] = jnp.zeros_like(out_ref)

    @pl.when(active)
    def _():
        x = xs_ref[...].astype(jnp.bfloat16)
        up = jnp.dot(x, wup_b[slot].astype(jnp.bfloat16),
                     preferred_element_type=jnp.float32)
        gate = jnp.dot(x, wgate_b[slot].astype(jnp.bfloat16),
                       preferred_element_type=jnp.float32)
        z = (up * jax.nn.sigmoid(up) * gate).astype(jnp.bfloat16)
        out_ref[...] = jnp.dot(
            z, wdown_b[slot].astype(jnp.bfloat16),
            preferred_element_type=jnp.float32
        ) * ws_ref[...][:, 0:1]


@functools.cache
def _make_scatter_sc():
    mesh = plsc.VectorSubcoreMesh(core_axis_name="c", subcore_axis_name="s")

    @functools.partial(
        pl.kernel,
        out_type=[
            jax.ShapeDtypeStruct((PAD_ROWS, D), jnp.float32),
            jax.ShapeDtypeStruct((PAD_ROWS, WREP), jnp.float32),
        ],
        mesh=mesh,
        scratch_types=[
            pltpu.VMEM((TPW,), jnp.int32),
            pltpu.VMEM((TPW,), jnp.int32),
            pltpu.VMEM((TPW, D), jnp.float32),
            pltpu.VMEM((TPW, WREP), jnp.float32),
            pltpu.VMEM((TPW, WREP), jnp.float32),
            pltpu.SemaphoreType.DMA,
        ],
    )
    def _scatter_sc(h_hbm, dst_hbm, wrep_hbm, xs_hbm, ws_hbm,
                    idx0_v, idx1_v, rows_v, w0_v, w1_v, sem):
        wid = lax.axis_index("s") * 2 + lax.axis_index("c")
        base = wid * TPW
        pltpu.sync_copy(dst_hbm.at[pl.ds(base, TPW)], idx0_v)
        pltpu.sync_copy(dst_hbm.at[pl.ds(S + base, TPW)], idx1_v)
        pltpu.sync_copy(h_hbm.at[pl.ds(base, TPW)], rows_v)
        pltpu.sync_copy(wrep_hbm.at[pl.ds(base, TPW)], w0_v)
        pltpu.sync_copy(wrep_hbm.at[pl.ds(S + base, TPW)], w1_v)
        d0 = pltpu.async_copy(rows_v, xs_hbm.at[idx0_v], sem)
        d1 = pltpu.async_copy(rows_v, xs_hbm.at[idx1_v], sem)
        d2 = pltpu.async_copy(w0_v, ws_hbm.at[idx0_v], sem)
        d3 = pltpu.async_copy(w1_v, ws_hbm.at[idx1_v], sem)
        d0.wait()
        d1.wait()
        d2.wait()
        d3.wait()

    return _scatter_sc


@functools.cache
def _make_combine_sc():
    mesh = plsc.VectorSubcoreMesh(core_axis_name="c", subcore_axis_name="s")

    @functools.partial(
        pl.kernel,
        out_type=jax.ShapeDtypeStruct((S, D), jnp.float32),
        mesh=mesh,
        scratch_types=[
            pltpu.VMEM((CH,), jnp.int32),
            pltpu.VMEM((CH,), jnp.int32),
            pltpu.VMEM((CH,), jnp.int32),
            pltpu.VMEM((CH,), jnp.int32),
            pltpu.VMEM((CH, D), jnp.float32),
            pltpu.VMEM((CH, D), jnp.float32),
            pltpu.VMEM((CH, D), jnp.float32),
            pltpu.VMEM((CH, D), jnp.float32),
            pltpu.SemaphoreType.DMA,
        ],
    )
    def _combine_sc(y_hbm, dst_hbm, out_hbm, i0, i1, i2, i3,
                    abuf, bbuf, cbuf, dbuf, sem):
        wid = lax.axis_index("s") * 2 + lax.axis_index("c")
        for ch in range(TPW // CH):
            base = wid * TPW + ch * CH
            pltpu.sync_copy(dst_hbm.at[pl.ds(base, CH)], i0)
            pltpu.sync_copy(dst_hbm.at[pl.ds(S + base, CH)], i1)
            pltpu.sync_copy(dst_hbm.at[pl.ds(2 * S + base, CH)], i2)
            pltpu.sync_copy(dst_hbm.at[pl.ds(3 * S + base, CH)], i3)
            d0 = pltpu.async_copy(y_hbm.at[i0], abuf, sem)
            d1 = pltpu.async_copy(y_hbm.at[i1], bbuf, sem)
            d2 = pltpu.async_copy(y_hbm.at[i2], cbuf, sem)
            d3 = pltpu.async_copy(y_hbm.at[i3], dbuf, sem)
            d0.wait()
            d1.wait()
            d2.wait()
            d3.wait()

            def row_body(r, _):
                def col_body(c, _):
                    av = abuf[r, pl.ds(c * LANES, LANES)]
                    bv = bbuf[r, pl.ds(c * LANES, LANES)]
                    cv = cbuf[r, pl.ds(c * LANES, LANES)]
                    dv = dbuf[r, pl.ds(c * LANES, LANES)]
                    abuf[r, pl.ds(c * LANES, LANES)] = (av + bv) + (cv + dv)
                    return 0

                lax.fori_loop(0, D // LANES, col_body, 0)
                return 0

            lax.fori_loop(0, CH, row_body, 0)
            pltpu.sync_copy(abuf, out_hbm.at[pl.ds(base, CH)])

    return _combine_sc


@jax.jit
def _run(h2d, Wg, W_up, W_gate, W_down):
    dst, wrep, texp, tact, tslot, tnext = pl.pallas_call(
        _router_meta_body,
        out_shape=[
            jax.ShapeDtypeStruct((2 * TOP_K * S, 1), jnp.int32),
            jax.ShapeDtypeStruct((TOP_K * S, WREP), jnp.float32),
            jax.ShapeDtypeStruct((1, MAXG), jnp.int32),
            jax.ShapeDtypeStruct((1, MAXG), jnp.int32),
            jax.ShapeDtypeStruct((1, MAXG), jnp.int32),
            jax.ShapeDtypeStruct((1, MAXG), jnp.int32),
        ],
    )(h2d, Wg)
    dst_flat = dst.reshape(2 * TOP_K * S)

    x_sorted, ws_sorted = _make_scatter_sc()(h2d, dst_flat, wrep)

    y = pl.pallas_call(
        _gmm_body,
        grid_spec=pltpu.PrefetchScalarGridSpec(
            num_scalar_prefetch=4,
            grid=(2, MAXG),
            in_specs=[
                pl.BlockSpec((T, D), lambda f, g, *_: (g, 0)),
                pl.BlockSpec((T, WREP), lambda f, g, *_: (g, 0)),
                pl.BlockSpec(memory_space=pltpu.MemorySpace.HBM),
                pl.BlockSpec(memory_space=pltpu.MemorySpace.HBM),
                pl.BlockSpec(memory_space=pltpu.MemorySpace.HBM),
            ],
            out_specs=pl.BlockSpec(
                (T, D), lambda f, g, *_: (f * MAXG + g, 0)
            ),
            scratch_shapes=[
                pltpu.VMEM((2, D, FH), jnp.float32),
                pltpu.VMEM((2, D, FH), jnp.float32),
                pltpu.VMEM((2, FH, D), jnp.float32),
                pltpu.SemaphoreType.DMA((3, 2)),
            ],
        ),
        out_shape=jax.ShapeDtypeStruct((2 * PAD_ROWS, D), jnp.float32),
    )(texp, tslot, tnext, tact, x_sorted, ws_sorted, W_up, W_gate, W_down)

    return _make_combine_sc()(y, dst_flat)


def kernel(hidden_states, Wg, W_up, W_gate, W_down):
    h2d = hidden_states.reshape(-1, D)
    out = _run(h2d, Wg, W_up, W_gate, W_down)
    return out.reshape(hidden_states.shape)


# speedup vs baseline: 1.9913x; 1.0400x over previous
import functools

import jax
import jax.numpy as jnp
from jax import lax
from jax.experimental import pallas as pl
from jax.experimental.pallas import tpu as pltpu
from jax.experimental.pallas import tpu_sc as plsc

B, S, D = 1, 2048, 1024
FF = 3584
E = 8
TOP_K = 2

T = 128
MAXG = (TOP_K * S) // T + E
PAD_ROWS = MAXG * T

NW = 32
TPW = S // NW
FH = FF // 2
CH = 32
LANES = 16
WREP = 128


def _router_meta_body(h_ref, wg_ref, dst_ref, wrep_ref, texp_ref, tact_ref,
                      tslot_ref, tnext_ref):
    h = h_ref[...]
    logits = jnp.dot(h, wg_ref[...], preferred_element_type=jnp.float32)
    m = jnp.max(logits, axis=1, keepdims=True)
    ex = jnp.exp(logits - m)
    p = ex / jnp.sum(ex, axis=1, keepdims=True)
    idx = lax.broadcasted_iota(jnp.int32, (S, E), 1)
    v0 = jnp.max(p, axis=1, keepdims=True)
    e0 = jnp.min(jnp.where(p == v0, idx, E), axis=1, keepdims=True)
    p1 = jnp.where(idx == e0, -jnp.inf, p)
    v1 = jnp.max(p1, axis=1, keepdims=True)
    e1 = jnp.min(jnp.where(p1 == v1, idx, E), axis=1, keepdims=True)
    s = v0 + v1
    w_a = jnp.concatenate([v0 / s, v1 / s], axis=0)
    wrep_ref[...] = jnp.broadcast_to(w_a, (TOP_K * S, WREP))

    e_a = jnp.concatenate([e0, e1], axis=0)
    idx2 = lax.broadcasted_iota(jnp.int32, (TOP_K * S, E), 1)
    oh = (e_a == idx2).astype(jnp.float32)

    RB = 512
    ri = lax.broadcasted_iota(jnp.int32, (RB, RB), 0)
    ci = lax.broadcasted_iota(jnp.int32, (RB, RB), 1)
    ltri = (ci < ri).astype(jnp.float32)
    carry = jnp.zeros((1, E), jnp.float32)
    ranks = []
    for b in range(TOP_K * S // RB):
        ohb = oh[b * RB:(b + 1) * RB, :]
        cb = jnp.dot(ltri, ohb, preferred_element_type=jnp.float32) + carry
        ranks.append(jnp.sum(cb * ohb, axis=1, keepdims=True))
        carry = carry + jnp.sum(ohb, axis=0, keepdims=True)
    rank_a = jnp.concatenate(ranks, axis=0)

    counts = carry
    ntiles = jnp.floor((counts + (T - 1)) / T)
    ri8 = lax.broadcasted_iota(jnp.int32, (E, E), 0)
    ci8 = lax.broadcasted_iota(jnp.int32, (E, E), 1)
    utri = (ri8 < ci8).astype(jnp.float32)
    cum_excl = jnp.dot(ntiles, utri, preferred_element_type=jnp.float32)
    poff = cum_excl * T
    poff_a = jnp.sum(oh * poff, axis=1, keepdims=True)
    dst_ref[...] = (rank_a + poff_a).astype(jnp.int32)

    ones_col = jnp.ones((TOP_K * S, 1), jnp.float32)
    counts_col = lax.dot_general(
        oh, ones_col, (((0,), (0,)), ((), ())),
        preferred_element_type=jnp.float32)
    ntiles_col = jnp.floor((counts_col + (T - 1)) / T)
    ltri8_inc = (ri8 >= ci8).astype(jnp.float32)
    cum_inc_col = jnp.dot(ltri8_inc, ntiles_col,
                          preferred_element_type=jnp.float32)
    cum_i = cum_inc_col.astype(jnp.int32)
    gi = lax.broadcasted_iota(jnp.int32, (E, MAXG), 1)
    texp_raw = jnp.sum((gi >= cum_i).astype(jnp.int32),
                       axis=0, keepdims=True)
    e_col = lax.broadcasted_iota(jnp.int32, (E, 1), 0)
    emax = jnp.max(jnp.where(ntiles_col > 0.5, e_col, -1))
    texp_row = jnp.minimum(texp_raw, emax)
    texp_ref[...] = texp_row
    total = jnp.max(cum_i)
    tact_ref[...] = (gi[0:1, :] < total).astype(jnp.int32)

    prev_row = jnp.concatenate([texp_row[:, :1], texp_row[:, :-1]], axis=1)
    c_run = jnp.logical_or(gi[0:1, :] == 0,
                           texp_row != prev_row).astype(jnp.float32)
    mi = lax.broadcasted_iota(jnp.int32, (MAXG, MAXG), 0)
    mj = lax.broadcasted_iota(jnp.int32, (MAXG, MAXG), 1)
    ltri_m = (mi <= mj).astype(jnp.float32)
    rid = jnp.dot(c_run, ltri_m, preferred_element_type=jnp.float32) - 1.0
    tslot_ref[...] = (rid - 2.0 * jnp.floor(rid * 0.5)).astype(jnp.int32)
    ecnt = jnp.broadcast_to(counts, (E, E))
    cand = jnp.where(jnp.logical_and(ci8 > ri8, ecnt > 0.5), ci8, 2 * E)
    nxt = jnp.min(cand, axis=1, keepdims=True)
    nxt = jnp.where(nxt == 2 * E, -1, nxt)
    e_col8b = lax.broadcasted_iota(jnp.int32, (E, 1), 0)
    onehot_t = jnp.broadcast_to(texp_row, (E, MAXG)) == e_col8b
    tnext_ref[...] = jnp.sum(
        jnp.where(onehot_t, jnp.broadcast_to(nxt, (E, MAXG)), 0),
        axis=0, keepdims=True).astype(jnp.int32)


def _gmm_body(fo, acc, texp_ref, tslot_ref, tnext_ref, tact_ref,
              xs_ref, ws_ref, *rest):
    if acc:
        (y0_ref, wup_hbm, wgate_hbm, wdown_hbm, out_ref,
         wup_b, wgate_b, wdown_b, sems) = rest
    else:
        (wup_hbm, wgate_hbm, wdown_hbm, out_ref,
         wup_b, wgate_b, wdown_b, sems) = rest
    g = pl.program_id(0)
    e = texp_ref[0, g]
    slot = tslot_ref[0, g]
    en = tnext_ref[0, g]
    active = tact_ref[0, g] == 1
    prev = texp_ref[0, jnp.maximum(g - 1, 0)]
    first = jnp.logical_or(g == 0, e != prev)

    def copies(eidx, sidx):
        return [
            pltpu.make_async_copy(wup_hbm.at[eidx, :, pl.ds(fo, FH)],
                                  wup_b.at[sidx], sems.at[0, sidx]),
            pltpu.make_async_copy(wgate_hbm.at[eidx, :, pl.ds(fo, FH)],
                                  wgate_b.at[sidx], sems.at[1, sidx]),
            pltpu.make_async_copy(wdown_hbm.at[eidx, pl.ds(fo, FH), :],
                                  wdown_b.at[sidx], sems.at[2, sidx]),
        ]

    @pl.when(g == 0)
    def _():
        for cp in copies(e, slot):
            cp.start()

    @pl.when(jnp.logical_and(first, en >= 0))
    def _():
        for cp in copies(en, 1 - slot):
            cp.start()

    @pl.when(first)
    def _():
        for cp in copies(e, slot):
            cp.wait()

    @pl.when(jnp.logical_not(active))
    def _():
        out_ref[...] = jnp.zeros_like(out_ref)

    @pl.when(active)
    def _():
        x = xs_ref[...].astype(jnp.bfloat16)
        up = jnp.dot(x, wup_b[slot].astype(jnp.bfloat16),
                     preferred_element_type=jnp.float32)
        gate = jnp.dot(x, wgate_b[slot].astype(jnp.bfloat16),
                       preferred_element_type=jnp.float32)
        z = (up * jax.nn.sigmoid(up) * gate).astype(jnp.bfloat16)
        part = jnp.dot(
            z, wdown_b[slot].astype(jnp.bfloat16),
            preferred_element_type=jnp.float32
        ) * ws_ref[...][:, 0:1]
        if acc:
            out_ref[...] = y0_ref[...] + part
        else:
            out_ref[...] = part


@functools.cache
def _make_scatter_sc():
    mesh = plsc.VectorSubcoreMesh(core_axis_name="c", subcore_axis_name="s")

    @functools.partial(
        pl.kernel,
        out_type=[
            jax.ShapeDtypeStruct((PAD_ROWS, D), jnp.float32),
            jax.ShapeDtypeStruct((PAD_ROWS, WREP), jnp.float32),
        ],
        mesh=mesh,
        scratch_types=[
            pltpu.VMEM((TPW,), jnp.int32),
            pltpu.VMEM((TPW,), jnp.int32),
            pltpu.VMEM((TPW, D), jnp.float32),
            pltpu.VMEM((TPW, WREP), jnp.float32),
            pltpu.VMEM((TPW, WREP), jnp.float32),
            pltpu.SemaphoreType.DMA,
        ],
    )
    def _scatter_sc(h_hbm, dst_hbm, wrep_hbm, xs_hbm, ws_hbm,
                    idx0_v, idx1_v, rows_v, w0_v, w1_v, sem):
        wid = lax.axis_index("s") * 2 + lax.axis_index("c")
        base = wid * TPW
        pltpu.sync_copy(dst_hbm.at[pl.ds(base, TPW)], idx0_v)
        pltpu.sync_copy(dst_hbm.at[pl.ds(S + base, TPW)], idx1_v)
        pltpu.sync_copy(h_hbm.at[pl.ds(base, TPW)], rows_v)
        pltpu.sync_copy(wrep_hbm.at[pl.ds(base, TPW)], w0_v)
        pltpu.sync_copy(wrep_hbm.at[pl.ds(S + base, TPW)], w1_v)
        d0 = pltpu.async_copy(rows_v, xs_hbm.at[idx0_v], sem)
        d1 = pltpu.async_copy(rows_v, xs_hbm.at[idx1_v], sem)
        d2 = pltpu.async_copy(w0_v, ws_hbm.at[idx0_v], sem)
        d3 = pltpu.async_copy(w1_v, ws_hbm.at[idx1_v], sem)
        d0.wait()
        d1.wait()
        d2.wait()
        d3.wait()

    return _scatter_sc


@functools.cache
def _make_combine_sc():
    mesh = plsc.VectorSubcoreMesh(core_axis_name="c", subcore_axis_name="s")

    @functools.partial(
        pl.kernel,
        out_type=jax.ShapeDtypeStruct((S, D), jnp.float32),
        mesh=mesh,
        scratch_types=[
            pltpu.VMEM((CH,), jnp.int32),
            pltpu.VMEM((CH,), jnp.int32),
            pltpu.VMEM((CH, D), jnp.float32),
            pltpu.VMEM((CH, D), jnp.float32),
            pltpu.SemaphoreType.DMA,
        ],
    )
    def _combine_sc(y_hbm, dst_hbm, out_hbm, i0, i1, abuf, bbuf, sem):
        wid = lax.axis_index("s") * 2 + lax.axis_index("c")
        for ch in range(TPW // CH):
            base = wid * TPW + ch * CH
            pltpu.sync_copy(dst_hbm.at[pl.ds(base, CH)], i0)
            pltpu.sync_copy(dst_hbm.at[pl.ds(S + base, CH)], i1)
            d0 = pltpu.async_copy(y_hbm.at[i0], abuf, sem)
            d1 = pltpu.async_copy(y_hbm.at[i1], bbuf, sem)
            d0.wait()
            d1.wait()

            def row_body(r, _):
                def col_body(c, _):
                    av = abuf[r, pl.ds(c * LANES, LANES)]
                    bv = bbuf[r, pl.ds(c * LANES, LANES)]
                    abuf[r, pl.ds(c * LANES, LANES)] = av + bv
                    return 0

                lax.fori_loop(0, D // LANES, col_body, 0)
                return 0

            lax.fori_loop(0, CH, row_body, 0)
            pltpu.sync_copy(abuf, out_hbm.at[pl.ds(base, CH)])

    return _combine_sc


@jax.jit
def _run(h2d, Wg, W_up, W_gate, W_down):
    dst, wrep, texp, tact, tslot, tnext = pl.pallas_call(
        _router_meta_body,
        out_shape=[
            jax.ShapeDtypeStruct((TOP_K * S, 1), jnp.int32),
            jax.ShapeDtypeStruct((TOP_K * S, WREP), jnp.float32),
            jax.ShapeDtypeStruct((1, MAXG), jnp.int32),
            jax.ShapeDtypeStruct((1, MAXG), jnp.int32),
            jax.ShapeDtypeStruct((1, MAXG), jnp.int32),
            jax.ShapeDtypeStruct((1, MAXG), jnp.int32),
        ],
    )(h2d, Wg)
    dst_flat = dst.reshape(TOP_K * S)

    x_sorted, ws_sorted = _make_scatter_sc()(h2d, dst_flat, wrep)

    def _gmm_spec(acc):
        in_specs = [
            pl.BlockSpec((T, D), lambda g, *_: (g, 0)),
            pl.BlockSpec((T, WREP), lambda g, *_: (g, 0)),
        ]
        if acc:
            in_specs.append(pl.BlockSpec((T, D), lambda g, *_: (g, 0)))
        in_specs += [
            pl.BlockSpec(memory_space=pltpu.MemorySpace.HBM),
            pl.BlockSpec(memory_space=pltpu.MemorySpace.HBM),
            pl.BlockSpec(memory_space=pltpu.MemorySpace.HBM),
        ]
        return pltpu.PrefetchScalarGridSpec(
            num_scalar_prefetch=4,
            grid=(MAXG,),
            in_specs=in_specs,
            out_specs=pl.BlockSpec((T, D), lambda g, *_: (g, 0)),
            scratch_shapes=[
                pltpu.VMEM((2, D, FH), jnp.float32),
                pltpu.VMEM((2, D, FH), jnp.float32),
                pltpu.VMEM((2, FH, D), jnp.float32),
                pltpu.SemaphoreType.DMA((3, 2)),
            ],
        )

    y_shape = jax.ShapeDtypeStruct((PAD_ROWS, D), jnp.float32)
    y0 = pl.pallas_call(
        functools.partial(_gmm_body, 0, False),
        grid_spec=_gmm_spec(False),
        out_shape=y_shape,
    )(texp, tslot, tnext, tact, x_sorted, ws_sorted, W_up, W_gate, W_down)
    y = pl.pallas_call(
        functools.partial(_gmm_body, FH, True),
        grid_spec=_gmm_spec(True),
        out_shape=y_shape,
    )(texp, tslot, tnext, tact, x_sorted, ws_sorted, y0,
      W_up, W_gate, W_down)

    return _make_combine_sc()(y, dst_flat)


def kernel(hidden_states, Wg, W_up, W_gate, W_down):
    h2d = hidden_states.reshape(-1, D)
    out = _run(h2d, Wg, W_up, W_gate, W_down)
    return out.reshape(hidden_states.shape)
